# baseline (speedup 1.0000x reference)
import functools

import jax
import jax.numpy as jnp
from jax import lax
from jax.experimental import pallas as pl
from jax.experimental.pallas import tpu as pltpu
from jax.experimental.pallas import tpu_sc as plsc

_K = 8192
_D = 256
_M = 8192
_BM = 256
_BN = 1024
_BC = 2048
_COMMIT = 0.25


def _argmin_body(xn_ref, en_ref, x_ref, e_ref, idx_ref, minv_ref):
    j = pl.program_id(1)

    @pl.when(j == 0)
    def _init():
        minv_ref[...] = jnp.full(minv_ref.shape, jnp.inf, jnp.float32)
        idx_ref[...] = jnp.zeros(idx_ref.shape, jnp.int32)

    mm = lax.dot_general(x_ref[...], e_ref[...], (((1,), (1,)), ((), ())),
                         preferred_element_type=jnp.float32)
    dist = (xn_ref[...] + en_ref[...]) - 2.0 * mm
    lmin = jnp.min(dist, axis=1, keepdims=True)
    cols = lax.broadcasted_iota(jnp.int32, dist.shape, 1) + j * _BN
    lidx = jnp.min(jnp.where(dist == lmin, cols, jnp.int32(2**30)),
                   axis=1, keepdims=True)
    better = lmin < minv_ref[...]
    idx_ref[...] = jnp.where(better, lidx, idx_ref[...])
    minv_ref[...] = jnp.where(better, lmin, minv_ref[...])


def _onehot_body(idx_ref, enc_ref, cnt_ref):
    i = pl.program_id(1)
    cols = lax.broadcasted_iota(jnp.int32, (_BM, _BC), 1) + pl.program_id(0) * _BC
    enc = (idx_ref[...] == cols).astype(jnp.float32)
    enc_ref[...] = enc

    @pl.when(i == 0)
    def _init():
        cnt_ref[...] = jnp.zeros(cnt_ref.shape, jnp.float32)

    cnt_ref[...] += jnp.sum(enc, axis=0, keepdims=True)


def _final_body(x_ref, q_ref, cnt_ref, qst_ref, loss_ref, perp_ref):
    x = x_ref[...]
    q = q_ref[...]
    d = q - x
    qst_ref[...] = x + d
    m = jnp.sum(d * d) / jnp.float32(_M * _D)
    loss_ref[...] = jnp.reshape(m + _COMMIT * m, (1, 1))
    p = cnt_ref[...] / jnp.float32(_M)
    ent = jnp.sum(p * jnp.log(p + 1e-10))
    perp_ref[...] = jnp.reshape(jnp.exp(-ent), (1, 1))


def _sc_gather(table, idx):
    nc, ns = 2, 16
    nw = nc * ns
    bpw = _M // nw
    half = bpw // 2
    mesh = plsc.VectorSubcoreMesh(core_axis_name="c", subcore_axis_name="s")

    @functools.partial(
        pl.kernel, mesh=mesh,
        out_type=jax.ShapeDtypeStruct((_M, _D), jnp.float32),
        scratch_types=[
            pltpu.VMEM((half,), jnp.int32),
            pltpu.VMEM((half,), jnp.int32),
            pltpu.VMEM((bpw, _D), jnp.float32),
            pltpu.SemaphoreType.DMA,
        ],
    )
    def gather_k(table_hbm, idx_hbm, out_hbm, idx_a, idx_b, rows_v, sem):
        wid = lax.axis_index("s") * nc + lax.axis_index("c")
        base = wid * bpw
        pltpu.sync_copy(idx_hbm.at[pl.ds(base, half)], idx_a)
        pltpu.sync_copy(idx_hbm.at[pl.ds(base + half, half)], idx_b)
        c0 = pltpu.async_copy(table_hbm.at[idx_a], rows_v.at[pl.ds(0, half)], sem)
        c1 = pltpu.async_copy(table_hbm.at[idx_b], rows_v.at[pl.ds(half, half)], sem)
        c0.wait()
        c1.wait()
        pltpu.sync_copy(rows_v, out_hbm.at[pl.ds(base, bpw)])

    return gather_k(table, idx)


def kernel(z, embedding):
    inputs = jnp.transpose(z, (0, 2, 3, 1))
    flat = inputs.reshape(_M, _D)
    xn = jnp.sum(flat ** 2, axis=1, keepdims=True)
    en = jnp.sum(embedding ** 2, axis=1)[None, :]

    idx = pl.pallas_call(
        _argmin_body,
        grid=(_M // _BM, _K // _BN),
        in_specs=[
            pl.BlockSpec((_BM, 1), lambda i, j: (i, 0)),
            pl.BlockSpec((1, _BN), lambda i, j: (0, j)),
            pl.BlockSpec((_BM, _D), lambda i, j: (i, 0)),
            pl.BlockSpec((_BN, _D), lambda i, j: (j, 0)),
        ],
        out_specs=pl.BlockSpec((_BM, 1), lambda i, j: (i, 0)),
        out_shape=jax.ShapeDtypeStruct((_M, 1), jnp.int32),
        scratch_shapes=[pltpu.VMEM((_BM, 1), jnp.float32)],
        compiler_params=pltpu.CompilerParams(
            dimension_semantics=("parallel", "arbitrary")),
    )(xn, en, flat, embedding)

    quantized = _sc_gather(embedding, idx.reshape(_M))

    encodings, counts = pl.pallas_call(
        _onehot_body,
        grid=(_K // _BC, _M // _BM),
        in_specs=[pl.BlockSpec((_BM, 1), lambda j, i: (i, 0))],
        out_specs=[
            pl.BlockSpec((_BM, _BC), lambda j, i: (i, j)),
            pl.BlockSpec((1, _BC), lambda j, i: (0, j)),
        ],
        out_shape=[
            jax.ShapeDtypeStruct((_M, _K), jnp.float32),
            jax.ShapeDtypeStruct((1, _K), jnp.float32),
        ],
        compiler_params=pltpu.CompilerParams(
            dimension_semantics=("arbitrary", "arbitrary")),
    )(idx)

    qst, loss, perp = pl.pallas_call(
        _final_body,
        out_shape=[
            jax.ShapeDtypeStruct((_M, _D), jnp.float32),
            jax.ShapeDtypeStruct((1, 1), jnp.float32),
            jax.ShapeDtypeStruct((1, 1), jnp.float32),
        ],
    )(flat, quantized, counts)

    q_out = jnp.transpose(qst.reshape(8, 32, 32, _D), (0, 3, 1, 2))
    return (loss[0, 0], q_out, perp[0, 0], encodings)


# speedup vs baseline: 2.2055x; 2.2055x over previous
import functools

import jax
import jax.numpy as jnp
from jax import lax
from jax.experimental import pallas as pl
from jax.experimental.pallas import tpu as pltpu
from jax.experimental.pallas import tpu_sc as plsc

_K = 8192
_D = 256
_M = 8192
_BM = 256
_COMMIT = 0.25


def _vq_body(xn_ref, en_ref, x_ref, e_ref, idx_ref, enc_ref, cnt_ref):
    i = pl.program_id(0)
    mm = lax.dot_general(x_ref[...], e_ref[...], (((1,), (1,)), ((), ())),
                         preferred_element_type=jnp.float32)
    dist = (xn_ref[...] + en_ref[...]) - 2.0 * mm
    lmin = jnp.min(dist, axis=1, keepdims=True)
    cols = lax.broadcasted_iota(jnp.int32, dist.shape, 1)
    idx = jnp.min(jnp.where(dist == lmin, cols, jnp.int32(2**30)),
                  axis=1, keepdims=True)
    idx_ref[...] = idx
    enc = (idx == cols).astype(jnp.float32)
    enc_ref[...] = enc

    @pl.when(i == 0)
    def _init():
        cnt_ref[...] = jnp.zeros(cnt_ref.shape, jnp.float32)

    cnt_ref[...] += jnp.sum(enc, axis=0, keepdims=True)


def _final_body(x_ref, q_ref, cnt_ref, qst_ref, loss_ref, perp_ref):
    x = x_ref[...]
    q = q_ref[...]
    d = q - x
    qst_ref[...] = x + d
    m = jnp.sum(d * d) / jnp.float32(_M * _D)
    loss_ref[...] = jnp.reshape(m + _COMMIT * m, (1, 1))
    p = cnt_ref[...] / jnp.float32(_M)
    ent = jnp.sum(p * jnp.log(p + 1e-10))
    perp_ref[...] = jnp.reshape(jnp.exp(-ent), (1, 1))


def _sc_gather(table, idx):
    nc, ns = 2, 16
    nw = nc * ns
    bpw = _M // nw
    half = bpw // 2
    mesh = plsc.VectorSubcoreMesh(core_axis_name="c", subcore_axis_name="s")

    @functools.partial(
        pl.kernel, mesh=mesh,
        out_type=jax.ShapeDtypeStruct((_M, _D), jnp.float32),
        scratch_types=[
            pltpu.VMEM((half,), jnp.int32),
            pltpu.VMEM((half,), jnp.int32),
            pltpu.VMEM((bpw, _D), jnp.float32),
            pltpu.SemaphoreType.DMA,
        ],
    )
    def gather_k(table_hbm, idx_hbm, out_hbm, idx_a, idx_b, rows_v, sem):
        wid = lax.axis_index("s") * nc + lax.axis_index("c")
        base = wid * bpw
        pltpu.sync_copy(idx_hbm.at[pl.ds(base, half)], idx_a)
        pltpu.sync_copy(idx_hbm.at[pl.ds(base + half, half)], idx_b)
        c0 = pltpu.async_copy(table_hbm.at[idx_a], rows_v.at[pl.ds(0, half)], sem)
        c1 = pltpu.async_copy(table_hbm.at[idx_b], rows_v.at[pl.ds(half, half)], sem)
        c0.wait()
        c1.wait()
        pltpu.sync_copy(rows_v, out_hbm.at[pl.ds(base, bpw)])

    return gather_k(table, idx)


def kernel(z, embedding):
    inputs = jnp.transpose(z, (0, 2, 3, 1))
    flat = inputs.reshape(_M, _D)
    xn = jnp.sum(flat ** 2, axis=1, keepdims=True)
    en = jnp.sum(embedding ** 2, axis=1)[None, :]

    idx, encodings, counts = pl.pallas_call(
        _vq_body,
        grid=(_M // _BM,),
        in_specs=[
            pl.BlockSpec((_BM, 1), lambda i: (i, 0)),
            pl.BlockSpec((1, _K), lambda i: (0, 0)),
            pl.BlockSpec((_BM, _D), lambda i: (i, 0)),
            pl.BlockSpec((_K, _D), lambda i: (0, 0)),
        ],
        out_specs=[
            pl.BlockSpec((_BM, 1), lambda i: (i, 0)),
            pl.BlockSpec((_BM, _K), lambda i: (i, 0)),
            pl.BlockSpec((1, _K), lambda i: (0, 0)),
        ],
        out_shape=[
            jax.ShapeDtypeStruct((_M, 1), jnp.int32),
            jax.ShapeDtypeStruct((_M, _K), jnp.float32),
            jax.ShapeDtypeStruct((1, _K), jnp.float32),
        ],
        compiler_params=pltpu.CompilerParams(
            dimension_semantics=("arbitrary",)),
    )(xn, en, flat, embedding)

    quantized = _sc_gather(embedding, idx.reshape(_M))

    qst, loss, perp = pl.pallas_call(
        _final_body,
        out_shape=[
            jax.ShapeDtypeStruct((_M, _D), jnp.float32),
            jax.ShapeDtypeStruct((1, 1), jnp.float32),
            jax.ShapeDtypeStruct((1, 1), jnp.float32),
        ],
    )(flat, quantized, counts)

    q_out = jnp.transpose(qst.reshape(8, 32, 32, _D), (0, 3, 1, 2))
    return (loss[0, 0], q_out, perp[0, 0], encodings)
